# baseline (speedup 1.0000x reference)
import functools

import jax
import jax.numpy as jnp
from jax import lax
from jax.experimental import pallas as pl
from jax.experimental.pallas import tpu as pltpu
from jax.experimental.pallas import tpu_sc as plsc

NS = 16
CHUNK = 64
BLK = 20
NSLOT = 5
BR = 256


def _mesh():
    return plsc.VectorSubcoreMesh(core_axis_name="c", subcore_axis_name="s",
                                  num_cores=1)


def _make_deg_kernel(n_pad, nchunk):
    rps = n_pad // NS

    @functools.partial(
        pl.kernel,
        out_type=jax.ShapeDtypeStruct((n_pad,), jnp.float32),
        mesh=_mesh(),
        scratch_types=[
            pltpu.VMEM((nchunk, CHUNK), jnp.int32),
            pltpu.VMEM((CHUNK,), jnp.float32),
            pltpu.VMEM_SHARED((n_pad,), jnp.float32),
            pltpu.SemaphoreType.DMA,
        ],
    )
    def deg_kernel(dst_hbm, zeros_hbm, ones_hbm, deg_hbm, idx_v, ones_v, acc_sh, sem):
        s = lax.axis_index("s")
        pltpu.sync_copy(zeros_hbm, acc_sh.at[pl.ds(s * rps, rps)])
        pltpu.sync_copy(ones_hbm, ones_v)
        pltpu.sync_copy(dst_hbm.at[s], idx_v)
        plsc.subcore_barrier()

        @pl.loop(0, nchunk)
        def _fire(j):
            pltpu.async_copy(ones_v, acc_sh.at[idx_v.at[j]], sem, add=True)

        @pl.loop(0, nchunk)
        def _drain(j):
            pltpu.make_async_copy(ones_v, acc_sh.at[idx_v.at[0]], sem).wait()

        plsc.subcore_barrier()
        pltpu.sync_copy(acc_sh.at[pl.ds(s * rps, rps)],
                        deg_hbm.at[pl.ds(s * rps, rps)])

    return deg_kernel


def _make_agg_kernel(n_pad, nchunk, d):
    rps = n_pad // NS
    nblk = nchunk // BLK

    @functools.partial(
        pl.kernel,
        out_type=jax.ShapeDtypeStruct((n_pad, d), jnp.float32),
        mesh=_mesh(),
        scratch_types=[
            pltpu.VMEM((BLK, CHUNK), jnp.int32),
            pltpu.VMEM((BLK, CHUNK), jnp.int32),
            pltpu.VMEM((NSLOT, CHUNK, d), jnp.float32),
            pltpu.VMEM_SHARED((n_pad, d), jnp.float32),
            pltpu.SemaphoreType.DMA((NSLOT,)),
            pltpu.SemaphoreType.DMA((NSLOT,)),
        ],
        compiler_params=pltpu.CompilerParams(use_tc_tiling_on_sc=False),
    )
    def agg_kernel(src_hbm, dst_hbm, y_hbm, out_hbm,
                   sblk, dblk, rows, acc_sh, gsems, ssems):
        s = lax.axis_index("s")

        def gather(j, slot, sem_slot):
            pltpu.async_copy(y_hbm.at[sblk.at[j]], rows.at[slot],
                             gsems.at[sem_slot])

        def gather_wait(slot):
            pltpu.make_async_copy(y_hbm.at[sblk.at[0]], rows.at[0],
                                  gsems.at[slot]).wait()

        def scatter(j, slot):
            pltpu.async_copy(rows.at[slot], acc_sh.at[dblk.at[j]],
                             ssems.at[slot], add=True)

        def scatter_wait(slot):
            pltpu.make_async_copy(rows.at[0], acc_sh.at[dblk.at[0]],
                                  ssems.at[slot]).wait()

        @pl.loop(0, CHUNK)
        def _zrow(i):
            @pl.loop(0, d // 16)
            def _zcol(k):
                rows[0, i, pl.ds(k * 16, 16)] = jnp.zeros((16,), jnp.float32)

        @pl.loop(0, rps // CHUNK)
        def _zinit(k):
            pltpu.sync_copy(rows.at[0],
                            acc_sh.at[pl.ds(s * rps + k * CHUNK, CHUNK)])

        plsc.subcore_barrier()

        @pl.loop(0, nblk)
        def _block(b):
            pltpu.sync_copy(src_hbm.at[s].at[pl.ds(b * BLK, BLK)], sblk)
            pltpu.sync_copy(dst_hbm.at[s].at[pl.ds(b * BLK, BLK)], dblk)

            gather(0, 0, 0)
            gather(1, 1, 1)
            gather(2, 2, 2)
            gather_wait(0)
            gather(3, 3, 3)
            scatter(0, 0)
            gather_wait(1)
            gather(4, 4, 4)
            scatter(1, 1)

            @pl.loop(2, BLK)
            def _body(jj):
                slot = lax.rem(jj, NSLOT)
                nslot = lax.rem(jj + 3, NSLOT)
                gather_wait(slot)
                scatter_wait(nslot)
                gather(jnp.minimum(jj + 3, BLK - 1), nslot, nslot)
                scatter(jj, slot)

            gather_wait(0)
            gather_wait(1)
            gather_wait(2)
            scatter_wait(3)
            scatter_wait(4)

        plsc.subcore_barrier()

        @pl.loop(0, rps // (2 * CHUNK))
        def _wb(k):
            r0 = s * rps + 2 * k * CHUNK
            pltpu.sync_copy(acc_sh.at[pl.ds(r0, CHUNK)], rows.at[0])
            pltpu.async_copy(rows.at[0], out_hbm.at[pl.ds(r0, CHUNK)],
                             gsems.at[0])
            pltpu.sync_copy(acc_sh.at[pl.ds(r0 + CHUNK, CHUNK)], rows.at[1])
            pltpu.async_copy(rows.at[1], out_hbm.at[pl.ds(r0 + CHUNK, CHUNK)],
                             gsems.at[1])
            pltpu.make_async_copy(rows.at[0], out_hbm.at[pl.ds(r0, CHUNK)],
                                  gsems.at[0]).wait()
            pltpu.make_async_copy(rows.at[1], out_hbm.at[pl.ds(r0, CHUNK)],
                                  gsems.at[1]).wait()

    return agg_kernel


def _mm1_body(deg_ref, x_ref, w_ref, y_ref, dinv_ref):
    dinv = lax.rsqrt(deg_ref[...] + 1.0)
    xw = jnp.dot(x_ref[...], w_ref[...], preferred_element_type=jnp.float32)
    y_ref[...] = xw * dinv
    dinv_ref[...] = dinv


def _mm2_body(a_ref, y_ref, dinv_ref, w_ref, b_ref, y2_ref):
    agg = a_ref[...] + y_ref[...]
    t = agg * dinv_ref[...] + b_ref[...]
    h = jnp.maximum(t, 0.0)
    y2_ref[...] = jnp.dot(h, w_ref[...], preferred_element_type=jnp.float32) * dinv_ref[...]


def _final_body(a_ref, y_ref, dinv_ref, b_ref, out_ref):
    agg = a_ref[...] + y_ref[...]
    t = agg * dinv_ref[...] + b_ref[...]
    m = jnp.max(t, axis=1, keepdims=True)
    lse = jnp.log(jnp.sum(jnp.exp(t - m), axis=1, keepdims=True)) + m
    out_ref[...] = t - lse


def _row_spec(d):
    return pl.BlockSpec((BR, d), lambda i: (i, 0))


def _full_spec(shape):
    return pl.BlockSpec(shape, lambda i: tuple(0 for _ in shape))


def kernel(features, edge_index, batch_size, W1, b1, W2, b2):
    n = features.shape[0]
    d_in = features.shape[1]
    d_hid = W1.shape[1]
    d_out = W2.shape[1]
    e = edge_index.shape[1]

    n_pad = ((n + 1 + BR - 1) // BR) * BR
    blk_edges = BLK * CHUNK
    per_tile = ((e + NS * blk_edges - 1) // (NS * blk_edges)) * blk_edges
    nchunk = per_tile // CHUNK
    ep = per_tile * NS

    src = edge_index[0].astype(jnp.int32)
    dst = edge_index[1].astype(jnp.int32)
    pad = ep - e
    src = jnp.concatenate([src, jnp.zeros((pad,), jnp.int32)])
    trash = n + jnp.arange(pad, dtype=jnp.int32) % (n_pad - n)
    dst = jnp.concatenate([dst, trash])
    src3 = src.reshape(NS, nchunk, CHUNK)
    dst3 = dst.reshape(NS, nchunk, CHUNK)

    x_pad = jnp.zeros((n_pad, d_in), features.dtype).at[:n].set(features)
    rps = n_pad // NS
    zeros_row = jnp.zeros((rps,), jnp.float32)
    ones_row = jnp.ones((CHUNK,), jnp.float32)

    grid = (n_pad // BR,)

    deg = _make_deg_kernel(n_pad, nchunk)(dst3, zeros_row, ones_row)

    y1, dinv = pl.pallas_call(
        _mm1_body,
        grid=grid,
        in_specs=[
            pl.BlockSpec((BR, 1), lambda i: (i, 0)),
            _row_spec(d_in),
            _full_spec((d_in, d_hid)),
        ],
        out_specs=[_row_spec(d_hid), pl.BlockSpec((BR, 1), lambda i: (i, 0))],
        out_shape=[
            jax.ShapeDtypeStruct((n_pad, d_hid), jnp.float32),
            jax.ShapeDtypeStruct((n_pad, 1), jnp.float32),
        ],
    )(deg.reshape(n_pad, 1), x_pad, W1)

    agg1 = _make_agg_kernel(n_pad, nchunk, d_hid)(src3, dst3, y1)

    y2 = pl.pallas_call(
        _mm2_body,
        grid=grid,
        in_specs=[
            _row_spec(d_hid),
            _row_spec(d_hid),
            pl.BlockSpec((BR, 1), lambda i: (i, 0)),
            _full_spec((d_hid, d_out)),
            _full_spec((1, d_hid)),
        ],
        out_specs=_row_spec(d_out),
        out_shape=jax.ShapeDtypeStruct((n_pad, d_out), jnp.float32),
    )(agg1, y1, dinv, W2, b1.reshape(1, d_hid))

    agg2 = _make_agg_kernel(n_pad, nchunk, d_out)(src3, dst3, y2)

    out = pl.pallas_call(
        _final_body,
        grid=grid,
        in_specs=[
            _row_spec(d_out),
            _row_spec(d_out),
            pl.BlockSpec((BR, 1), lambda i: (i, 0)),
            _full_spec((1, d_out)),
        ],
        out_specs=_row_spec(d_out),
        out_shape=jax.ShapeDtypeStruct((n_pad, d_out), jnp.float32),
    )(agg2, y2, dinv, b2.reshape(1, d_out))

    return out[:n]


# speedup vs baseline: 1.1265x; 1.1265x over previous
import functools

import jax
import jax.numpy as jnp
from jax import lax
from jax.experimental import pallas as pl
from jax.experimental.pallas import tpu as pltpu
from jax.experimental.pallas import tpu_sc as plsc

NC = 2
NS = 16
CHUNK = 64
BLK = 20
NSLOT = 5
BR = 256


def _mesh():
    return plsc.VectorSubcoreMesh(core_axis_name="c", subcore_axis_name="s")


def _make_deg_kernel(n_pad, nchunk):
    rps = n_pad // NS

    @functools.partial(
        pl.kernel,
        out_type=jax.ShapeDtypeStruct((NC, n_pad), jnp.float32),
        mesh=_mesh(),
        scratch_types=[
            pltpu.VMEM((nchunk, CHUNK), jnp.int32),
            pltpu.VMEM((CHUNK,), jnp.float32),
            pltpu.VMEM_SHARED((n_pad,), jnp.float32),
            pltpu.SemaphoreType.DMA,
        ],
    )
    def deg_kernel(dst_hbm, zeros_hbm, ones_hbm, deg_hbm, idx_v, ones_v, acc_sh, sem):
        c = lax.axis_index("c")
        s = lax.axis_index("s")
        nchunk_t = jnp.where(c == 0, nchunk, 0)
        pltpu.sync_copy(zeros_hbm, acc_sh.at[pl.ds(s * rps, rps)])
        pltpu.sync_copy(ones_hbm, ones_v)
        pltpu.sync_copy(dst_hbm.at[s], idx_v)
        plsc.subcore_barrier()

        @pl.loop(0, nchunk_t)
        def _fire(j):
            pltpu.async_copy(ones_v, acc_sh.at[idx_v.at[j]], sem, add=True)

        @pl.loop(0, nchunk_t)
        def _drain(j):
            pltpu.make_async_copy(ones_v, acc_sh.at[idx_v.at[0]], sem).wait()

        plsc.subcore_barrier()
        pltpu.sync_copy(acc_sh.at[pl.ds(s * rps, rps)],
                        deg_hbm.at[c].at[pl.ds(s * rps, rps)])

    return deg_kernel


def _make_agg_kernel(n_pad, nchunk, d):
    rps = n_pad // NS
    nblk = nchunk // BLK

    @functools.partial(
        pl.kernel,
        out_type=jax.ShapeDtypeStruct((NC, n_pad, d), jnp.float32),
        mesh=_mesh(),
        scratch_types=[
            pltpu.VMEM((BLK, CHUNK), jnp.int32),
            pltpu.VMEM((BLK, CHUNK), jnp.int32),
            pltpu.VMEM((NSLOT, CHUNK, d), jnp.float32),
            pltpu.VMEM_SHARED((n_pad, d), jnp.float32),
            pltpu.SemaphoreType.DMA((NSLOT,)),
            pltpu.SemaphoreType.DMA((NSLOT,)),
        ],
        compiler_params=pltpu.CompilerParams(use_tc_tiling_on_sc=False),
    )
    def agg_kernel(src_hbm, dst_hbm, y_hbm, out_hbm,
                   sblk, dblk, rows, acc_sh, gsems, ssems):
        c = lax.axis_index("c")
        s = lax.axis_index("s")
        nblk_t = jnp.where(c == 0, nblk, 0)

        def gather(j, slot, sem_slot):
            pltpu.async_copy(y_hbm.at[sblk.at[j]], rows.at[slot],
                             gsems.at[sem_slot])

        def gather_wait(slot):
            pltpu.make_async_copy(y_hbm.at[sblk.at[0]], rows.at[0],
                                  gsems.at[slot]).wait()

        def scatter(j, slot):
            pltpu.async_copy(rows.at[slot], acc_sh.at[dblk.at[j]],
                             ssems.at[slot], add=True)

        def scatter_wait(slot):
            pltpu.make_async_copy(rows.at[0], acc_sh.at[dblk.at[0]],
                                  ssems.at[slot]).wait()

        @pl.loop(0, CHUNK)
        def _zrow(i):
            @pl.loop(0, d // 16)
            def _zcol(k):
                rows[0, i, pl.ds(k * 16, 16)] = jnp.zeros((16,), jnp.float32)

        @pl.loop(0, rps // CHUNK)
        def _zinit(k):
            pltpu.sync_copy(rows.at[0],
                            acc_sh.at[pl.ds(s * rps + k * CHUNK, CHUNK)])

        plsc.subcore_barrier()

        @pl.loop(0, nblk_t)
        def _block(b):
            pltpu.sync_copy(src_hbm.at[s].at[pl.ds(b * BLK, BLK)], sblk)
            pltpu.sync_copy(dst_hbm.at[s].at[pl.ds(b * BLK, BLK)], dblk)

            gather(0, 0, 0)
            gather(1, 1, 1)
            gather(2, 2, 2)
            gather_wait(0)
            gather(3, 3, 3)
            scatter(0, 0)
            gather_wait(1)
            gather(4, 4, 4)
            scatter(1, 1)

            @pl.loop(2, BLK)
            def _body(jj):
                slot = lax.rem(jj, NSLOT)
                nslot = lax.rem(jj + 3, NSLOT)
                gather_wait(slot)
                scatter_wait(nslot)
                gather(jnp.minimum(jj + 3, BLK - 1), nslot, nslot)
                scatter(jj, slot)

            gather_wait(0)
            gather_wait(1)
            gather_wait(2)
            scatter_wait(3)
            scatter_wait(4)

        plsc.subcore_barrier()

        @pl.loop(0, rps // (2 * CHUNK))
        def _wb(k):
            r0 = s * rps + 2 * k * CHUNK
            pltpu.sync_copy(acc_sh.at[pl.ds(r0, CHUNK)], rows.at[0])
            pltpu.async_copy(rows.at[0], out_hbm.at[c].at[pl.ds(r0, CHUNK)],
                             gsems.at[0])
            pltpu.sync_copy(acc_sh.at[pl.ds(r0 + CHUNK, CHUNK)], rows.at[1])
            pltpu.async_copy(rows.at[1],
                             out_hbm.at[c].at[pl.ds(r0 + CHUNK, CHUNK)],
                             gsems.at[1])
            pltpu.make_async_copy(rows.at[0], out_hbm.at[c].at[pl.ds(r0, CHUNK)],
                                  gsems.at[0]).wait()
            pltpu.make_async_copy(rows.at[1], out_hbm.at[c].at[pl.ds(r0, CHUNK)],
                                  gsems.at[1]).wait()

    return agg_kernel


def _mm1_body(deg_ref, x_ref, w_ref, y_ref, dinv_ref):
    dinv = lax.rsqrt(deg_ref[:, 0:1] + deg_ref[:, 1:2] + 1.0)
    xw = jnp.dot(x_ref[...], w_ref[...], preferred_element_type=jnp.float32)
    y_ref[...] = xw * dinv
    dinv_ref[...] = dinv


def _mm2_body(a_ref, y_ref, dinv_ref, w_ref, b_ref, y2_ref):
    agg = a_ref[0] + a_ref[1] + y_ref[...]
    t = agg * dinv_ref[...] + b_ref[...]
    h = jnp.maximum(t, 0.0)
    y2_ref[...] = jnp.dot(h, w_ref[...], preferred_element_type=jnp.float32) * dinv_ref[...]


def _final_body(a_ref, y_ref, dinv_ref, b_ref, out_ref):
    agg = a_ref[0] + a_ref[1] + y_ref[...]
    t = agg * dinv_ref[...] + b_ref[...]
    m = jnp.max(t, axis=1, keepdims=True)
    lse = jnp.log(jnp.sum(jnp.exp(t - m), axis=1, keepdims=True)) + m
    out_ref[...] = t - lse


def _row_spec(d):
    return pl.BlockSpec((BR, d), lambda i: (i, 0))


def _full_spec(shape):
    return pl.BlockSpec(shape, lambda i: tuple(0 for _ in shape))


def _agg_spec(d):
    return pl.BlockSpec((NC, BR, d), lambda i: (0, i, 0))


def kernel(features, edge_index, batch_size, W1, b1, W2, b2):
    n = features.shape[0]
    d_in = features.shape[1]
    d_hid = W1.shape[1]
    d_out = W2.shape[1]
    e = edge_index.shape[1]

    n_pad = ((n + 1 + BR - 1) // BR) * BR
    blk_edges = BLK * CHUNK
    per_tile = ((e + NS * blk_edges - 1) // (NS * blk_edges)) * blk_edges
    nchunk = per_tile // CHUNK
    ep = per_tile * NS

    src = edge_index[0].astype(jnp.int32)
    dst = edge_index[1].astype(jnp.int32)
    pad = ep - e
    src = jnp.concatenate([src, jnp.zeros((pad,), jnp.int32)])
    trash = n + jnp.arange(pad, dtype=jnp.int32) % (n_pad - n)
    dst = jnp.concatenate([dst, trash])
    src3 = src.reshape(NS, nchunk, CHUNK)
    dst3 = dst.reshape(NS, nchunk, CHUNK)

    x_pad = jnp.zeros((n_pad, d_in), features.dtype).at[:n].set(features)
    rps = n_pad // NS
    zeros_row = jnp.zeros((rps,), jnp.float32)
    ones_row = jnp.ones((CHUNK,), jnp.float32)

    grid = (n_pad // BR,)

    deg = _make_deg_kernel(n_pad, nchunk)(dst3, zeros_row, ones_row)

    y1, dinv = pl.pallas_call(
        _mm1_body,
        grid=grid,
        in_specs=[
            pl.BlockSpec((BR, NC), lambda i: (i, 0)),
            _row_spec(d_in),
            _full_spec((d_in, d_hid)),
        ],
        out_specs=[_row_spec(d_hid), pl.BlockSpec((BR, 1), lambda i: (i, 0))],
        out_shape=[
            jax.ShapeDtypeStruct((n_pad, d_hid), jnp.float32),
            jax.ShapeDtypeStruct((n_pad, 1), jnp.float32),
        ],
    )(deg.T, x_pad, W1)

    agg1 = _make_agg_kernel(n_pad, nchunk, d_hid)(src3, dst3, y1)

    y2 = pl.pallas_call(
        _mm2_body,
        grid=grid,
        in_specs=[
            _agg_spec(d_hid),
            _row_spec(d_hid),
            pl.BlockSpec((BR, 1), lambda i: (i, 0)),
            _full_spec((d_hid, d_out)),
            _full_spec((1, d_hid)),
        ],
        out_specs=_row_spec(d_out),
        out_shape=jax.ShapeDtypeStruct((n_pad, d_out), jnp.float32),
    )(agg1, y1, dinv, W2, b1.reshape(1, d_hid))

    agg2 = _make_agg_kernel(n_pad, nchunk, d_out)(src3, dst3, y2)

    out = pl.pallas_call(
        _final_body,
        grid=grid,
        in_specs=[
            _agg_spec(d_out),
            _row_spec(d_out),
            pl.BlockSpec((BR, 1), lambda i: (i, 0)),
            _full_spec((1, d_out)),
        ],
        out_specs=_row_spec(d_out),
        out_shape=jax.ShapeDtypeStruct((n_pad, d_out), jnp.float32),
    )(agg2, y2, dinv, b2.reshape(1, d_out))

    return out[:n]


# speedup vs baseline: 1.4141x; 1.2553x over previous
import functools

import jax
import jax.numpy as jnp
from jax import lax
from jax.experimental import pallas as pl
from jax.experimental.pallas import tpu as pltpu
from jax.experimental.pallas import tpu_sc as plsc

NC = 2
NS = 16
NW = NC * NS
CHUNK = 64
BLK = 40
NSLOT = 4
BR = 256
B0_RATIO = 3


def _mesh():
    return plsc.VectorSubcoreMesh(core_axis_name="c", subcore_axis_name="s")


def _make_deg_kernel(n_pad, nblk0, nblk1):
    rps = n_pad // NS

    @functools.partial(
        pl.kernel,
        out_type=jax.ShapeDtypeStruct((NC, n_pad), jnp.float32),
        mesh=_mesh(),
        scratch_types=[
            pltpu.VMEM((nblk0 * BLK, CHUNK), jnp.int32),
            pltpu.VMEM((CHUNK,), jnp.float32),
            pltpu.VMEM_SHARED((n_pad,), jnp.float32),
            pltpu.SemaphoreType.DMA,
        ],
    )
    def deg_kernel(dst_hbm, zeros_hbm, ones_hbm, deg_hbm, idx_v, ones_v, acc_sh, sem):
        c = lax.axis_index("c")
        s = lax.axis_index("s")
        w = c * NS + s
        nchunk_t = jnp.where(c == 0, nblk0 * BLK, nblk1 * BLK)
        pltpu.sync_copy(zeros_hbm, acc_sh.at[pl.ds(s * rps, rps)])
        pltpu.sync_copy(ones_hbm, ones_v)
        pltpu.sync_copy(dst_hbm.at[w], idx_v)
        plsc.subcore_barrier()

        @pl.loop(0, nchunk_t)
        def _fire(j):
            pltpu.async_copy(ones_v, acc_sh.at[idx_v.at[j]], sem, add=True)

        @pl.loop(0, nchunk_t)
        def _drain(j):
            pltpu.make_async_copy(ones_v, acc_sh.at[idx_v.at[0]], sem).wait()

        plsc.subcore_barrier()
        pltpu.sync_copy(acc_sh.at[pl.ds(s * rps, rps)],
                        deg_hbm.at[c].at[pl.ds(s * rps, rps)])

    return deg_kernel


def _make_agg_kernel(n_pad, nblk0, nblk1, d):
    rps = n_pad // NS

    @functools.partial(
        pl.kernel,
        out_type=jax.ShapeDtypeStruct((NC, n_pad, d), jnp.float32),
        mesh=_mesh(),
        scratch_types=[
            pltpu.VMEM((BLK, CHUNK), jnp.int32),
            pltpu.VMEM((BLK, CHUNK), jnp.int32),
            pltpu.VMEM((NSLOT, CHUNK, d), jnp.float32),
            pltpu.VMEM_SHARED((n_pad, d), jnp.float32),
            pltpu.SemaphoreType.DMA((NSLOT,)),
            pltpu.SemaphoreType.DMA((NSLOT,)),
        ],
        compiler_params=pltpu.CompilerParams(use_tc_tiling_on_sc=False),
    )
    def agg_kernel(src_hbm, dst_hbm, y_hbm, zeros_hbm, out_hbm,
                   sblk, dblk, rows, acc_sh, gsems, ssems):
        c = lax.axis_index("c")
        s = lax.axis_index("s")
        w = c * NS + s
        nblk_t = jnp.where(c == 0, nblk0, nblk1)

        def gather(j, slot, sem_slot):
            pltpu.async_copy(y_hbm.at[sblk.at[j]], rows.at[slot],
                             gsems.at[sem_slot])

        def gather_wait(slot):
            pltpu.make_async_copy(y_hbm.at[sblk.at[0]], rows.at[0],
                                  gsems.at[slot]).wait()

        def scatter(j, slot):
            pltpu.async_copy(rows.at[slot], acc_sh.at[dblk.at[j]],
                             ssems.at[slot], add=True)

        def scatter_wait(slot):
            pltpu.make_async_copy(rows.at[0], acc_sh.at[dblk.at[0]],
                                  ssems.at[slot]).wait()

        pltpu.sync_copy(zeros_hbm, acc_sh.at[pl.ds(s * rps, rps)])
        plsc.subcore_barrier()

        @pl.loop(0, nblk_t)
        def _block(b):
            pltpu.sync_copy(src_hbm.at[w].at[pl.ds(b * BLK, BLK)], sblk)
            pltpu.sync_copy(dst_hbm.at[w].at[pl.ds(b * BLK, BLK)], dblk)

            gather(0, 0, 0)
            gather(1, 1, 1)
            gather_wait(0)
            gather(2, 2, 2)
            scatter(0, 0)
            gather_wait(1)
            gather(3, 3, 3)
            scatter(1, 1)

            @pl.loop(2, BLK)
            def _body(jj):
                slot = lax.rem(jj, NSLOT)
                nslot = lax.rem(jj + 2, NSLOT)
                gather_wait(slot)
                scatter_wait(nslot)
                gather(jnp.minimum(jj + 2, BLK - 1), nslot, nslot)
                scatter(jj, slot)

            gather_wait(0)
            gather_wait(1)
            scatter_wait(2)
            scatter_wait(3)

        plsc.subcore_barrier()
        pltpu.sync_copy(acc_sh.at[pl.ds(s * rps, rps)],
                        out_hbm.at[c].at[pl.ds(s * rps, rps)])

    return agg_kernel


def _mm1_body(deg_ref, x_ref, w_ref, y_ref, dinv_ref):
    d = deg_ref[:, 0:1] + deg_ref[:, 1:2] + 1.0
    dinv = lax.rsqrt(d)
    xw = jnp.dot(x_ref[...], w_ref[...], preferred_element_type=jnp.float32)
    y_ref[...] = xw * dinv
    dinv_ref[...] = dinv


def _mm2_body(a_ref, y_ref, dinv_ref, w_ref, b_ref, y2_ref):
    agg = a_ref[0] + a_ref[1] + y_ref[...]
    t = agg * dinv_ref[...] + b_ref[...]
    h = jnp.maximum(t, 0.0)
    y2_ref[...] = jnp.dot(h, w_ref[...], preferred_element_type=jnp.float32) * dinv_ref[...]


def _final_body(a_ref, y_ref, dinv_ref, b_ref, out_ref):
    agg = a_ref[0] + a_ref[1] + y_ref[...]
    t = agg * dinv_ref[...] + b_ref[...]
    m = jnp.max(t, axis=1, keepdims=True)
    lse = jnp.log(jnp.sum(jnp.exp(t - m), axis=1, keepdims=True)) + m
    out_ref[...] = t - lse


def _row_spec(d):
    return pl.BlockSpec((BR, d), lambda i: (i, 0))


def _full_spec(shape):
    return pl.BlockSpec(shape, lambda i: tuple(0 for _ in shape))


def _agg_spec(d):
    return pl.BlockSpec((NC, BR, d), lambda i: (0, i, 0))


def kernel(features, edge_index, batch_size, W1, b1, W2, b2):
    n = features.shape[0]
    d_in = features.shape[1]
    d_hid = W1.shape[1]
    d_out = W2.shape[1]
    e = edge_index.shape[1]

    n_pad = ((n + 1 + BR - 1) // BR) * BR
    blk_edges = BLK * CHUNK
    blocks_needed = (e + blk_edges - 1) // blk_edges
    u = (blocks_needed + NS * (B0_RATIO + 1) - 1) // (NS * (B0_RATIO + 1))
    nblk0, nblk1 = B0_RATIO * u, u
    nchunk_max = nblk0 * BLK
    ep = NS * (nblk0 + nblk1) * blk_edges

    src = edge_index[0].astype(jnp.int32)
    dst = edge_index[1].astype(jnp.int32)
    pad = ep - e
    src = jnp.concatenate([src, jnp.zeros((pad,), jnp.int32)])
    dst = jnp.concatenate([dst, jnp.full((pad,), n, jnp.int32)])
    e0 = NS * nblk0 * blk_edges

    def to3(a):
        a0 = a[:e0].reshape(NS, nchunk_max, CHUNK)
        a1 = a[e0:].reshape(NS, nblk1 * BLK, CHUNK)
        a1 = jnp.pad(a1, ((0, 0), (0, (nblk0 - nblk1) * BLK), (0, 0)),
                     constant_values=n)
        return jnp.concatenate([a0, a1], axis=0)

    src3 = to3(src)
    dst3 = to3(dst)

    x_pad = jnp.zeros((n_pad, d_in), features.dtype).at[:n].set(features)
    rps = n_pad // NS
    zeros_row = jnp.zeros((rps,), jnp.float32)
    ones_row = jnp.ones((CHUNK,), jnp.float32)
    zeros_hid = jnp.zeros((rps, d_hid), jnp.float32)
    zeros_out = jnp.zeros((rps, d_out), jnp.float32)

    grid = (n_pad // BR,)

    deg = _make_deg_kernel(n_pad, nblk0, nblk1)(dst3, zeros_row, ones_row)
    deg_t = deg.T

    y1, dinv = pl.pallas_call(
        _mm1_body,
        grid=grid,
        in_specs=[
            pl.BlockSpec((BR, NC), lambda i: (i, 0)),
            _row_spec(d_in),
            _full_spec((d_in, d_hid)),
        ],
        out_specs=[_row_spec(d_hid), pl.BlockSpec((BR, 1), lambda i: (i, 0))],
        out_shape=[
            jax.ShapeDtypeStruct((n_pad, d_hid), jnp.float32),
            jax.ShapeDtypeStruct((n_pad, 1), jnp.float32),
        ],
    )(deg_t, x_pad, W1)

    agg1 = _make_agg_kernel(n_pad, nblk0, nblk1, d_hid)(src3, dst3, y1, zeros_hid)

    y2 = pl.pallas_call(
        _mm2_body,
        grid=grid,
        in_specs=[
            _agg_spec(d_hid),
            _row_spec(d_hid),
            pl.BlockSpec((BR, 1), lambda i: (i, 0)),
            _full_spec((d_hid, d_out)),
            _full_spec((1, d_hid)),
        ],
        out_specs=_row_spec(d_out),
        out_shape=jax.ShapeDtypeStruct((n_pad, d_out), jnp.float32),
    )(agg1, y1, dinv, W2, b1.reshape(1, d_hid))

    agg2 = _make_agg_kernel(n_pad, nblk0, nblk1, d_out)(src3, dst3, y2, zeros_out)

    out = pl.pallas_call(
        _final_body,
        grid=grid,
        in_specs=[
            _agg_spec(d_out),
            _row_spec(d_out),
            pl.BlockSpec((BR, 1), lambda i: (i, 0)),
            _full_spec((1, d_out)),
        ],
        out_specs=_row_spec(d_out),
        out_shape=jax.ShapeDtypeStruct((n_pad, d_out), jnp.float32),
    )(agg2, y2, dinv, b2.reshape(1, d_out))

    return out[:n]
